# Initial kernel scaffold; baseline (speedup 1.0000x reference)
import jax
import jax.numpy as jnp
from jax.experimental import pallas as pl


def kernel(node_embeddings, edge_index, edge_weight, node_features):
    raise NotImplementedError("write your pallas kernel here")


# speedup vs baseline: 5.1652x; 5.1652x over previous
import functools

import jax
import jax.numpy as jnp
from jax import lax
from jax.experimental import pallas as pl
from jax.experimental.pallas import tpu as pltpu
from jax.experimental.pallas import tpu_sc as plsc

N = 10000
E = 320000
H = 128
NF6 = 6
OUT_D = 264

NC = 2
NS = 16
NW = NC * NS
EPW = E // NW
B = 80
NCHUNK = EPW // B
L = 16


def _rsqrt(x):
    xi = lax.bitcast_convert_type(x, jnp.int32)
    yi = jnp.int32(0x5F3759DF) - lax.shift_right_logical(xi, 1)
    y = lax.bitcast_convert_type(yi, jnp.float32)
    xh = x * 0.5
    for _ in range(3):
        y = y * (1.5 - xh * y * y)
    return y


def _edge_body(row_hbm, col_hbm, weight_hbm, emb_hbm, feat6_hbm, out_hbm,
               idx_row, idx_col, wv, emb_a, emb_b, feat6, sfeat,
               sem_emb):
    wid = lax.axis_index("s") * NC + lax.axis_index("c")
    pltpu.sync_copy(feat6_hbm, feat6)

    def chunk(g, carry):
        base = wid * EPW + g * B
        pltpu.sync_copy(row_hbm.at[pl.ds(base, B)], idx_row)
        pltpu.sync_copy(col_hbm.at[pl.ds(base, B)], idx_col)
        pltpu.sync_copy(weight_hbm.at[pl.ds(base, B)], wv)
        cp_a = pltpu.async_copy(emb_hbm.at[idx_row], emb_a, sem_emb)
        cp_b = pltpu.async_copy(emb_hbm.at[idx_col], emb_b, sem_emb)

        for grp in range(B // L):
            e0 = grp * L
            ni = idx_row[pl.ds(e0, L)] * NF6
            nj = idx_col[pl.ds(e0, L)] * NF6

            def gcol(nidx, c):
                return plsc.load_gather(feat6, [nidx + c])

            fa = [gcol(ni, c) for c in range(6)]
            fb = [gcol(nj, c) for c in range(6)]
            dot = fa[0] * fb[0] + fa[1] * fb[1] + fa[2] * fb[2] + fa[3] * fb[3]
            si = fa[0] * fa[0] + fa[1] * fa[1] + fa[2] * fa[2] + fa[3] * fa[3]
            sj = fb[0] * fb[0] + fb[1] * fb[1] + fb[2] * fb[2] + fb[3] * fb[3]
            sim = dot * _rsqrt(jnp.maximum(si, 1e-16)) * _rsqrt(jnp.maximum(sj, 1e-16))
            dx = fa[4] - fb[4]
            dy = fa[5] - fb[5]
            r = _rsqrt(dx * dx + dy * dy + 1e-12)
            w = jnp.abs(wv[pl.ds(e0, L)])
            vals = [w, sim, dx, dy, jnp.abs(dx), jnp.abs(dy), dx * r, dy * r]
            ei = lax.iota(jnp.int32, L) + e0
            for k, v in enumerate(vals):
                kk = jnp.full((L,), k, jnp.int32)
                plsc.store_scatter(sfeat, [ei, kk], v)

        cp_a.wait()
        cp_b.wait()
        pltpu.sync_copy(emb_a, out_hbm.at[pl.ds(base, B), pl.ds(0, H)])
        pltpu.sync_copy(emb_b, out_hbm.at[pl.ds(base, B), pl.ds(H, H)])
        pltpu.sync_copy(sfeat, out_hbm.at[pl.ds(base, B), pl.ds(2 * H, 8)])
        return carry

    lax.fori_loop(0, NCHUNK, chunk, 0)


@jax.jit
def _encode(node_embeddings, row, col, edge_weight, feat6):
    mesh = plsc.VectorSubcoreMesh(core_axis_name="c", subcore_axis_name="s")
    k = pl.kernel(
        _edge_body,
        out_type=jax.ShapeDtypeStruct((E, OUT_D), jnp.float32),
        mesh=mesh,
        scratch_types=[
            pltpu.VMEM((B,), jnp.int32),
            pltpu.VMEM((B,), jnp.int32),
            pltpu.VMEM((B,), jnp.float32),
            pltpu.VMEM((B, H), jnp.float32),
            pltpu.VMEM((B, H), jnp.float32),
            pltpu.VMEM((N * NF6,), jnp.float32),
            pltpu.VMEM((B, 8), jnp.float32),
            pltpu.SemaphoreType.DMA,
        ],
        compiler_params=pltpu.CompilerParams(needs_layout_passes=False),
    )
    return k(row, col, edge_weight, node_embeddings, feat6)


def kernel(node_embeddings, edge_index, edge_weight, node_features):
    row = edge_index[0]
    col = edge_index[1]
    feat6 = node_features[:, :NF6].reshape(-1)
    return _encode(node_embeddings, row, col, edge_weight, feat6)


# speedup vs baseline: 5.9121x; 1.1446x over previous
import functools

import jax
import jax.numpy as jnp
from jax import lax
from jax.experimental import pallas as pl
from jax.experimental.pallas import tpu as pltpu
from jax.experimental.pallas import tpu_sc as plsc

N = 10000
E = 320000
H = 128
NF6 = 6
OUT_D = 264

NC = 2
NS = 16
NW = NC * NS
EPW = E // NW
B = 80
NCHUNK = EPW // B
L = 16


def _rsqrt(x):
    xi = lax.bitcast_convert_type(x, jnp.int32)
    yi = jnp.int32(0x5F3759DF) - lax.shift_right_logical(xi, 1)
    y = lax.bitcast_convert_type(yi, jnp.float32)
    xh = x * 0.5
    for _ in range(3):
        y = y * (1.5 - xh * y * y)
    return y


def _edge_body(row_hbm, col_hbm, weight_hbm, emb_hbm, feat6_hbm, out_hbm,
               idx_row, idx_col, wv, feat6, pack0, pack1,
               sem_emb, sem_out0, sem_out1):
    wid = lax.axis_index("s") * NC + lax.axis_index("c")
    pltpu.sync_copy(feat6_hbm, feat6)

    def chunk(g, pack, sem_out, drain):
        base = wid * EPW + g * B
        if drain:
            pltpu.make_async_copy(
                out_hbm.at[pl.ds(0, B)], pack, sem_out).wait()
        pltpu.sync_copy(row_hbm.at[pl.ds(base, B)], idx_row)
        pltpu.sync_copy(col_hbm.at[pl.ds(base, B)], idx_col)
        pltpu.sync_copy(weight_hbm.at[pl.ds(base, B)], wv)
        cp_a = pltpu.async_copy(
            emb_hbm.at[idx_row], pack.at[:, pl.ds(0, H)], sem_emb)
        cp_b = pltpu.async_copy(
            emb_hbm.at[idx_col], pack.at[:, pl.ds(H, H)], sem_emb)

        for grp in range(B // L):
            e0 = grp * L
            ni = idx_row[pl.ds(e0, L)] * NF6
            nj = idx_col[pl.ds(e0, L)] * NF6

            def gcol(nidx, c):
                return plsc.load_gather(feat6, [nidx + c])

            fa = [gcol(ni, c) for c in range(6)]
            fb = [gcol(nj, c) for c in range(6)]
            dot = fa[0] * fb[0] + fa[1] * fb[1] + fa[2] * fb[2] + fa[3] * fb[3]
            si = fa[0] * fa[0] + fa[1] * fa[1] + fa[2] * fa[2] + fa[3] * fa[3]
            sj = fb[0] * fb[0] + fb[1] * fb[1] + fb[2] * fb[2] + fb[3] * fb[3]
            sim = dot * _rsqrt(jnp.maximum(si, 1e-16)) * _rsqrt(jnp.maximum(sj, 1e-16))
            dx = fa[4] - fb[4]
            dy = fa[5] - fb[5]
            r = _rsqrt(dx * dx + dy * dy + 1e-12)
            w = jnp.abs(wv[pl.ds(e0, L)])
            vals = [w, sim, dx, dy, jnp.abs(dx), jnp.abs(dy), dx * r, dy * r]
            ei = lax.iota(jnp.int32, L) + e0
            for k, v in enumerate(vals):
                kk = jnp.full((L,), 2 * H + k, jnp.int32)
                plsc.store_scatter(pack, [ei, kk], v)

        cp_a.wait()
        cp_b.wait()
        pltpu.async_copy(pack, out_hbm.at[pl.ds(base, B)], sem_out)

    chunk(0, pack0, sem_out0, drain=False)
    chunk(1, pack1, sem_out1, drain=False)

    def pair(p, carry):
        chunk(2 * p, pack0, sem_out0, drain=True)
        chunk(2 * p + 1, pack1, sem_out1, drain=True)
        return carry

    lax.fori_loop(1, NCHUNK // 2, pair, 0)
    chunk(NCHUNK - 1, pack0, sem_out0, drain=True)

    pltpu.make_async_copy(out_hbm.at[pl.ds(0, B)], pack1, sem_out1).wait()
    pltpu.make_async_copy(out_hbm.at[pl.ds(0, B)], pack0, sem_out0).wait()


@jax.jit
def _encode(node_embeddings, row, col, edge_weight, feat6):
    mesh = plsc.VectorSubcoreMesh(core_axis_name="c", subcore_axis_name="s")
    k = pl.kernel(
        _edge_body,
        out_type=jax.ShapeDtypeStruct((E, OUT_D), jnp.float32),
        mesh=mesh,
        scratch_types=[
            pltpu.VMEM((B,), jnp.int32),
            pltpu.VMEM((B,), jnp.int32),
            pltpu.VMEM((B,), jnp.float32),
            pltpu.VMEM((N * NF6,), jnp.float32),
            pltpu.VMEM((B, OUT_D), jnp.float32),
            pltpu.VMEM((B, OUT_D), jnp.float32),
            pltpu.SemaphoreType.DMA,
            pltpu.SemaphoreType.DMA,
            pltpu.SemaphoreType.DMA,
        ],
        compiler_params=pltpu.CompilerParams(needs_layout_passes=False),
    )
    return k(row, col, edge_weight, node_embeddings, feat6)


def kernel(node_embeddings, edge_index, edge_weight, node_features):
    row = edge_index[0]
    col = edge_index[1]
    feat6 = node_features[:, :NF6].reshape(-1)
    return _encode(node_embeddings, row, col, edge_weight, feat6)


# speedup vs baseline: 6.9440x; 1.1745x over previous
import functools

import jax
import jax.numpy as jnp
from jax import lax
from jax.experimental import pallas as pl
from jax.experimental.pallas import tpu as pltpu
from jax.experimental.pallas import tpu_sc as plsc

N = 10000
E = 320000
H = 128
NF6 = 6
OUT_D = 264

NC = 2
NS = 16
NW = NC * NS
EPW = E // NW
B = 80
NCHUNK = EPW // B
L = 16


def _rsqrt(x):
    xi = lax.bitcast_convert_type(x, jnp.int32)
    yi = jnp.int32(0x5F3759DF) - lax.shift_right_logical(xi, 1)
    y = lax.bitcast_convert_type(yi, jnp.float32)
    xh = x * 0.5
    for _ in range(3):
        y = y * (1.5 - xh * y * y)
    return y


def _edge_body(row_hbm, col_hbm, weight_hbm, emb_hbm, feat6_hbm, out_hbm,
               ir0, ir1, ic0, ic1, wv0, wv1, feat6, pack0, pack1,
               sem_emb, sem_out0, sem_out1, sem_i0, sem_i1):
    wid = lax.axis_index("s") * NC + lax.axis_index("c")
    base0 = wid * EPW
    pltpu.sync_copy(feat6_hbm, feat6)

    def prefetch(g, ir, ic, wv, sem):
        b = jnp.minimum(base0 + g * B, base0 + EPW - B)
        pltpu.async_copy(row_hbm.at[pl.ds(b, B)], ir, sem)
        pltpu.async_copy(col_hbm.at[pl.ds(b, B)], ic, sem)
        pltpu.async_copy(weight_hbm.at[pl.ds(b, B)], wv, sem)

    def drain_prefetch(ir, ic, wv, sem):
        pltpu.make_async_copy(row_hbm.at[pl.ds(0, B)], ir, sem).wait()
        pltpu.make_async_copy(col_hbm.at[pl.ds(0, B)], ic, sem).wait()
        pltpu.make_async_copy(weight_hbm.at[pl.ds(0, B)], wv, sem).wait()

    def chunk(g, pack, sem_out, ir, ic, wv, sem_i, ir_n, ic_n, wv_n, sem_i_n,
              drain):
        base = base0 + g * B
        if drain:
            pltpu.make_async_copy(
                out_hbm.at[pl.ds(0, B)], pack, sem_out).wait()
        drain_prefetch(ir, ic, wv, sem_i)
        cp_a = pltpu.async_copy(
            emb_hbm.at[ir], pack.at[:, pl.ds(0, H)], sem_emb)
        cp_b = pltpu.async_copy(
            emb_hbm.at[ic], pack.at[:, pl.ds(H, H)], sem_emb)
        prefetch(g + 1, ir_n, ic_n, wv_n, sem_i_n)

        for grp in range(B // L):
            e0 = grp * L
            ni = ir[pl.ds(e0, L)] * NF6
            nj = ic[pl.ds(e0, L)] * NF6

            def gcol(nidx, c):
                return plsc.load_gather(feat6, [nidx + c])

            fa = [gcol(ni, c) for c in range(6)]
            fb = [gcol(nj, c) for c in range(6)]
            dot = fa[0] * fb[0] + fa[1] * fb[1] + fa[2] * fb[2] + fa[3] * fb[3]
            si = fa[0] * fa[0] + fa[1] * fa[1] + fa[2] * fa[2] + fa[3] * fa[3]
            sj = fb[0] * fb[0] + fb[1] * fb[1] + fb[2] * fb[2] + fb[3] * fb[3]
            sim = dot * _rsqrt(jnp.maximum(si, 1e-16)) * _rsqrt(jnp.maximum(sj, 1e-16))
            dx = fa[4] - fb[4]
            dy = fa[5] - fb[5]
            r = _rsqrt(dx * dx + dy * dy + 1e-12)
            w = jnp.abs(wv[pl.ds(e0, L)])
            vals = [w, sim, dx, dy, jnp.abs(dx), jnp.abs(dy), dx * r, dy * r]
            ei = lax.iota(jnp.int32, L) + e0
            for k, v in enumerate(vals):
                kk = jnp.full((L,), 2 * H + k, jnp.int32)
                plsc.store_scatter(pack, [ei, kk], v)

        cp_a.wait()
        cp_b.wait()
        pltpu.async_copy(pack, out_hbm.at[pl.ds(base, B)], sem_out)

    prefetch(0, ir0, ic0, wv0, sem_i0)

    chunk(0, pack0, sem_out0, ir0, ic0, wv0, sem_i0,
          ir1, ic1, wv1, sem_i1, drain=False)
    chunk(1, pack1, sem_out1, ir1, ic1, wv1, sem_i1,
          ir0, ic0, wv0, sem_i0, drain=False)

    def pair(p, carry):
        chunk(2 * p, pack0, sem_out0, ir0, ic0, wv0, sem_i0,
              ir1, ic1, wv1, sem_i1, drain=True)
        chunk(2 * p + 1, pack1, sem_out1, ir1, ic1, wv1, sem_i1,
              ir0, ic0, wv0, sem_i0, drain=True)
        return carry

    lax.fori_loop(1, NCHUNK // 2, pair, 0)
    chunk(NCHUNK - 1, pack0, sem_out0, ir0, ic0, wv0, sem_i0,
          ir1, ic1, wv1, sem_i1, drain=True)

    pltpu.make_async_copy(out_hbm.at[pl.ds(0, B)], pack1, sem_out1).wait()
    pltpu.make_async_copy(out_hbm.at[pl.ds(0, B)], pack0, sem_out0).wait()
    drain_prefetch(ir1, ic1, wv1, sem_i1)


@jax.jit
def _encode(node_embeddings, row, col, edge_weight, feat6):
    mesh = plsc.VectorSubcoreMesh(core_axis_name="c", subcore_axis_name="s")
    k = pl.kernel(
        _edge_body,
        out_type=jax.ShapeDtypeStruct((E, OUT_D), jnp.float32),
        mesh=mesh,
        scratch_types=[
            pltpu.VMEM((B,), jnp.int32),
            pltpu.VMEM((B,), jnp.int32),
            pltpu.VMEM((B,), jnp.int32),
            pltpu.VMEM((B,), jnp.int32),
            pltpu.VMEM((B,), jnp.float32),
            pltpu.VMEM((B,), jnp.float32),
            pltpu.VMEM((N * NF6,), jnp.float32),
            pltpu.VMEM((B, OUT_D), jnp.float32),
            pltpu.VMEM((B, OUT_D), jnp.float32),
            pltpu.SemaphoreType.DMA,
            pltpu.SemaphoreType.DMA,
            pltpu.SemaphoreType.DMA,
            pltpu.SemaphoreType.DMA,
            pltpu.SemaphoreType.DMA,
        ],
        compiler_params=pltpu.CompilerParams(needs_layout_passes=False),
    )
    return k(row, col, edge_weight, node_embeddings, feat6)


def kernel(node_embeddings, edge_index, edge_weight, node_features):
    row = edge_index[0]
    col = edge_index[1]
    feat6 = node_features[:, :NF6].reshape(-1)
    return _encode(node_embeddings, row, col, edge_weight, feat6)
